# Initial kernel scaffold; baseline (speedup 1.0000x reference)
import functools

import jax
import jax.numpy as jnp
from jax import lax
from jax.experimental import pallas as pl
from jax.experimental.pallas import tpu as pltpu
from jax.experimental.pallas import tpu_sc as plsc

NC = 2
NS = 16
CH = 128
BLK = 8


def _round_up(x, m):
    return (x + m - 1) // m * m


def _mesh():
    return plsc.VectorSubcoreMesh(
        core_axis_name="c", subcore_axis_name="s",
        num_cores=NC, num_subcores=NS)


def _sc_embed(z_pad, emb0, emb1):
    n_pad, h = z_pad.shape[0], emb0.shape[1]
    chunks = n_pad // (NS * CH)

    @functools.partial(
        pl.kernel,
        out_type=[jax.ShapeDtypeStruct((n_pad, h), jnp.float32),
                  jax.ShapeDtypeStruct((n_pad, h), jnp.float32)],
        mesh=_mesh(),
        scratch_types=[pltpu.VMEM((CH,), jnp.int32),
                       pltpu.VMEM((CH, h), jnp.float32),
                       pltpu.SemaphoreType.DMA],
    )
    def k(z_ref, e0_ref, e1_ref, x0_ref, x1_ref, idx_v, rows_v, sem):
        c = lax.axis_index("c")
        s = lax.axis_index("s")
        base = s * (chunks * CH)

        def body(e_ref, x_ref):
            @pl.loop(0, chunks)
            def _(j):
                off = base + j * CH
                pltpu.sync_copy(z_ref.at[pl.ds(off, CH)], idx_v)
                pltpu.async_copy(e_ref.at[idx_v], rows_v, sem).wait()
                pltpu.sync_copy(rows_v, x_ref.at[pl.ds(off, CH)])

        @pl.when(c == 0)
        def _():
            body(e0_ref, x0_ref)

        @pl.when(c == 1)
        def _():
            body(e1_ref, x1_ref)

    return k(z_pad, emb0, emb1)


def _sc_agg(x0, x1, row2d, col2d, zrs):
    n, h = x0.shape
    rps = row2d.shape[0] // NS
    nb = rps // BLK
    zper = zrs.shape[0]
    sp_rows = zper * NS
    nout = n // NS

    @functools.partial(
        pl.kernel,
        out_type=[jax.ShapeDtypeStruct((n, h), jnp.float32),
                  jax.ShapeDtypeStruct((n, h), jnp.float32)],
        mesh=_mesh(),
        scratch_types=[pltpu.VMEM((BLK, CH), jnp.int32),
                       pltpu.VMEM((BLK, CH), jnp.int32),
                       pltpu.VMEM((BLK, CH, h), jnp.float32),
                       pltpu.VMEM_SHARED((sp_rows, h), jnp.float32),
                       pltpu.SemaphoreType.DMA,
                       pltpu.SemaphoreType.DMA],
    )
    def k(x0_ref, x1_ref, row_ref, col_ref, z_ref, o0_ref, o1_ref,
          rowbuf, colbuf, gbuf, acc, sem_g, sem_s):
        c = lax.axis_index("c")
        s = lax.axis_index("s")

        pltpu.sync_copy(z_ref, acc.at[pl.ds(s * zper, zper)])
        plsc.subcore_barrier()

        def body(x_ref, o_ref):
            @pl.loop(0, nb)
            def _(b):
                r0 = s * rps + b * BLK
                pltpu.sync_copy(row_ref.at[pl.ds(r0, BLK)], rowbuf)
                pltpu.sync_copy(col_ref.at[pl.ds(r0, BLK)], colbuf)
                gds = [pltpu.async_copy(x_ref.at[colbuf.at[j]],
                                        gbuf.at[j], sem_g)
                       for j in range(BLK)]
                sds = []
                for j in range(BLK):
                    gds[j].wait()
                    sds.append(pltpu.async_copy(
                        gbuf.at[j], acc.at[rowbuf.at[j]], sem_s, add=True))
                for dsc in sds:
                    dsc.wait()

            plsc.subcore_barrier()
            pltpu.sync_copy(acc.at[pl.ds(s * nout, nout)],
                            o_ref.at[pl.ds(s * nout, nout)])

        @pl.when(c == 0)
        def _():
            body(x0_ref, o0_ref)

        @pl.when(c == 1)
        def _():
            body(x1_ref, o1_ref)

    return k(x0, x1, row2d, col2d, zrs)


def _tc_linear(x0, x1, a0, a1, w0t, w1t, b2d, final):
    n, h = x0.shape
    d = w0t.shape[1]
    rb = 1000
    grid = (n // rb,)

    def compute(x0r, x1r, a0r, a1r, w0r, w1r, br):
        y = (jnp.dot(x0r[...] + a0r[...], w0r[...],
                     preferred_element_type=jnp.float32)
             + jnp.dot(x1r[...] + a1r[...], w1r[...],
                       preferred_element_type=jnp.float32)
             + br[...])
        return jnp.maximum(y, 0.0)

    in_specs = (
        [pl.BlockSpec((rb, h), lambda i: (i, 0)) for _ in range(4)]
        + [pl.BlockSpec((h, d), lambda i: (0, 0)) for _ in range(2)]
        + [pl.BlockSpec((1, d), lambda i: (0, 0))]
    )

    if final:
        def body(x0r, x1r, a0r, a1r, w0r, w1r, br, o):
            o[...] = compute(x0r, x1r, a0r, a1r, w0r, w1r, br)

        return pl.pallas_call(
            body, grid=grid, in_specs=in_specs,
            out_specs=pl.BlockSpec((rb, d), lambda i: (i, 0)),
            out_shape=jax.ShapeDtypeStruct((n, d), jnp.float32),
        )(x0, x1, a0, a1, w0t, w1t, b2d)

    def body2(x0r, x1r, a0r, a1r, w0r, w1r, br, o0, o1):
        y = compute(x0r, x1r, a0r, a1r, w0r, w1r, br)
        o0[...] = y[:, :h]
        o1[...] = y[:, h:]

    return pl.pallas_call(
        body2, grid=grid, in_specs=in_specs,
        out_specs=[pl.BlockSpec((rb, h), lambda i: (i, 0)),
                   pl.BlockSpec((rb, h), lambda i: (i, 0))],
        out_shape=[jax.ShapeDtypeStruct((n, h), jnp.float32),
                   jax.ShapeDtypeStruct((n, h), jnp.float32)],
    )(x0, x1, a0, a1, w0t, w1t, b2d)


def kernel(z, edge_index, emb, W0, b0, W1, b1, W2, b2):
    n = z.shape[0]
    e = edge_index.shape[1]
    d = emb.shape[1]
    h = d // 2

    n_pad = _round_up(n, NS * CH)
    z_pad = jnp.concatenate(
        [z.astype(jnp.int32), jnp.zeros((n_pad - n,), jnp.int32)])
    x0p, x1p = _sc_embed(z_pad, emb[:, :h], emb[:, h:])
    x0, x1 = x0p[:n], x1p[:n]

    eps = _round_up(-(-e // NS), BLK * CH)
    e_pad = eps * NS
    row = edge_index[0].astype(jnp.int32)
    col = edge_index[1].astype(jnp.int32)
    row2d = jnp.concatenate(
        [row, jnp.full((e_pad - e,), n, jnp.int32)]).reshape(-1, CH)
    col2d = jnp.concatenate(
        [col, jnp.zeros((e_pad - e,), jnp.int32)]).reshape(-1, CH)
    sp_rows = _round_up(n + 1, NS)
    zrs = jnp.zeros((sp_rows // NS, h), jnp.float32)

    for i, (w, b) in enumerate(((W0, b0), (W1, b1), (W2, b2))):
        a0, a1 = _sc_agg(x0, x1, row2d, col2d, zrs)
        wt = w.T
        if i < 2:
            x0, x1 = _tc_linear(x0, x1, a0, a1, wt[:h], wt[h:],
                                b.reshape(1, d), final=False)
        else:
            return _tc_linear(x0, x1, a0, a1, wt[:h], wt[h:],
                              b.reshape(1, d), final=True)


# speedup vs baseline: 6.9562x; 6.9562x over previous
import functools

import jax
import jax.numpy as jnp
from jax import lax
from jax.experimental import pallas as pl
from jax.experimental.pallas import tpu as pltpu
from jax.experimental.pallas import tpu_sc as plsc

NC = 2
NS = 16
CH = 128
BLK = 4


def _round_up(x, m):
    return (x + m - 1) // m * m


def _mesh():
    return plsc.VectorSubcoreMesh(
        core_axis_name="c", subcore_axis_name="s",
        num_cores=NC, num_subcores=NS)


def _sc_embed(z_pad, emb0, emb1):
    n_pad, h = z_pad.shape[0], emb0.shape[1]
    chunks = n_pad // (NS * CH)

    @functools.partial(
        pl.kernel,
        out_type=[jax.ShapeDtypeStruct((n_pad, h), jnp.float32),
                  jax.ShapeDtypeStruct((n_pad, h), jnp.float32)],
        mesh=_mesh(),
        compiler_params=pltpu.CompilerParams(use_tc_tiling_on_sc=False),
        scratch_types=[pltpu.VMEM((CH,), jnp.int32),
                       pltpu.VMEM((CH, h), jnp.float32),
                       pltpu.SemaphoreType.DMA],
    )
    def k(z_ref, e0_ref, e1_ref, x0_ref, x1_ref, idx_v, rows_v, sem):
        c = lax.axis_index("c")
        s = lax.axis_index("s")
        base = s * (chunks * CH)

        def body(e_ref, x_ref):
            @pl.loop(0, chunks)
            def _(j):
                off = base + j * CH
                pltpu.sync_copy(z_ref.at[pl.ds(off, CH)], idx_v)
                pltpu.async_copy(e_ref.at[idx_v], rows_v, sem).wait()
                pltpu.sync_copy(rows_v, x_ref.at[pl.ds(off, CH)])

        @pl.when(c == 0)
        def _():
            body(e0_ref, x0_ref)

        @pl.when(c == 1)
        def _():
            body(e1_ref, x1_ref)

    return k(z_pad, emb0, emb1)


def _sc_agg(x0, x1, row2d, col2d, zrs):
    n, h = x0.shape
    rps = row2d.shape[0] // NS
    nb = rps // BLK
    zper = zrs.shape[0]
    sp_rows = zper * NS
    nout = n // NS

    @functools.partial(
        pl.kernel,
        out_type=[jax.ShapeDtypeStruct((n, h), jnp.float32),
                  jax.ShapeDtypeStruct((n, h), jnp.float32)],
        mesh=_mesh(),
        compiler_params=pltpu.CompilerParams(use_tc_tiling_on_sc=False),
        scratch_types=[pltpu.VMEM((BLK, CH), jnp.int32),
                       pltpu.VMEM((BLK, CH), jnp.int32),
                       pltpu.VMEM((BLK, CH, h), jnp.float32),
                       pltpu.VMEM_SHARED((sp_rows, h), jnp.float32),
                       pltpu.SemaphoreType.DMA,
                       pltpu.SemaphoreType.DMA],
    )
    def k(x0_ref, x1_ref, row_ref, col_ref, z_ref, o0_ref, o1_ref,
          rowbuf, colbuf, gbuf, acc, sem_g, sem_s):
        c = lax.axis_index("c")
        s = lax.axis_index("s")

        pltpu.sync_copy(z_ref, acc.at[pl.ds(s * zper, zper)])
        plsc.subcore_barrier()

        def body(x_ref, o_ref):
            @pl.loop(0, nb)
            def _(b):
                r0 = s * rps + b * BLK
                pltpu.sync_copy(row_ref.at[pl.ds(r0, BLK)], rowbuf)
                pltpu.sync_copy(col_ref.at[pl.ds(r0, BLK)], colbuf)
                gds = [pltpu.async_copy(x_ref.at[colbuf.at[j]],
                                        gbuf.at[j], sem_g)
                       for j in range(BLK)]
                for dsc in gds:
                    dsc.wait()
                sds = [pltpu.async_copy(
                    gbuf.at[j], acc.at[rowbuf.at[j]], sem_s, add=True)
                       for j in range(BLK)]
                for dsc in sds:
                    dsc.wait()

            plsc.subcore_barrier()
            pltpu.sync_copy(acc.at[pl.ds(s * nout, nout)],
                            o_ref.at[pl.ds(s * nout, nout)])

        @pl.when(c == 0)
        def _():
            body(x0_ref, o0_ref)

        @pl.when(c == 1)
        def _():
            body(x1_ref, o1_ref)

    return k(x0, x1, row2d, col2d, zrs)


def _tc_linear(x0, x1, a0, a1, w0t, w1t, b2d, final):
    n, h = x0.shape
    d = w0t.shape[1]
    rb = 1000
    grid = (n // rb,)

    def compute(x0r, x1r, a0r, a1r, w0r, w1r, br):
        y = (jnp.dot(x0r[...] + a0r[...], w0r[...],
                     preferred_element_type=jnp.float32)
             + jnp.dot(x1r[...] + a1r[...], w1r[...],
                       preferred_element_type=jnp.float32)
             + br[...])
        return jnp.maximum(y, 0.0)

    in_specs = (
        [pl.BlockSpec((rb, h), lambda i: (i, 0)) for _ in range(4)]
        + [pl.BlockSpec((h, d), lambda i: (0, 0)) for _ in range(2)]
        + [pl.BlockSpec((1, d), lambda i: (0, 0))]
    )

    if final:
        def body(x0r, x1r, a0r, a1r, w0r, w1r, br, o):
            o[...] = compute(x0r, x1r, a0r, a1r, w0r, w1r, br)

        return pl.pallas_call(
            body, grid=grid, in_specs=in_specs,
            out_specs=pl.BlockSpec((rb, d), lambda i: (i, 0)),
            out_shape=jax.ShapeDtypeStruct((n, d), jnp.float32),
        )(x0, x1, a0, a1, w0t, w1t, b2d)

    def body2(x0r, x1r, a0r, a1r, w0r, w1r, br, o0, o1):
        y = compute(x0r, x1r, a0r, a1r, w0r, w1r, br)
        o0[...] = y[:, :h]
        o1[...] = y[:, h:]

    return pl.pallas_call(
        body2, grid=grid, in_specs=in_specs,
        out_specs=[pl.BlockSpec((rb, h), lambda i: (i, 0)),
                   pl.BlockSpec((rb, h), lambda i: (i, 0))],
        out_shape=[jax.ShapeDtypeStruct((n, h), jnp.float32),
                   jax.ShapeDtypeStruct((n, h), jnp.float32)],
    )(x0, x1, a0, a1, w0t, w1t, b2d)


def kernel(z, edge_index, emb, W0, b0, W1, b1, W2, b2):
    n = z.shape[0]
    e = edge_index.shape[1]
    d = emb.shape[1]
    h = d // 2

    n_pad = _round_up(n, NS * CH)
    z_pad = jnp.concatenate(
        [z.astype(jnp.int32), jnp.zeros((n_pad - n,), jnp.int32)])
    x0p, x1p = _sc_embed(z_pad, emb[:, :h], emb[:, h:])
    x0, x1 = x0p[:n], x1p[:n]

    eps = _round_up(-(-e // NS), BLK * CH)
    e_pad = eps * NS
    row = edge_index[0].astype(jnp.int32)
    col = edge_index[1].astype(jnp.int32)
    row2d = jnp.concatenate(
        [row, jnp.full((e_pad - e,), n, jnp.int32)]).reshape(-1, CH)
    col2d = jnp.concatenate(
        [col, jnp.zeros((e_pad - e,), jnp.int32)]).reshape(-1, CH)
    sp_rows = _round_up(n + 1, NS)
    zrs = jnp.zeros((sp_rows // NS, h), jnp.float32)

    for i, (w, b) in enumerate(((W0, b0), (W1, b1), (W2, b2))):
        a0, a1 = _sc_agg(x0, x1, row2d, col2d, zrs)
        wt = w.T
        if i < 2:
            x0, x1 = _tc_linear(x0, x1, a0, a1, wt[:h], wt[h:],
                                b.reshape(1, d), final=False)
        else:
            return _tc_linear(x0, x1, a0, a1, wt[:h], wt[h:],
                              b.reshape(1, d), final=True)


# speedup vs baseline: 7.0465x; 1.0130x over previous
import functools

import jax
import jax.numpy as jnp
from jax import lax
from jax.experimental import pallas as pl
from jax.experimental.pallas import tpu as pltpu
from jax.experimental.pallas import tpu_sc as plsc

NC = 2
NS = 16
CH = 128
BLK = 4


def _round_up(x, m):
    return (x + m - 1) // m * m


def _mesh():
    return plsc.VectorSubcoreMesh(
        core_axis_name="c", subcore_axis_name="s",
        num_cores=NC, num_subcores=NS)


def _sc_embed(z_pad, emb0, emb1):
    n_pad, h = z_pad.shape[0], emb0.shape[1]
    chunks = n_pad // (NS * CH)

    @functools.partial(
        pl.kernel,
        out_type=[jax.ShapeDtypeStruct((n_pad, h), jnp.float32),
                  jax.ShapeDtypeStruct((n_pad, h), jnp.float32)],
        mesh=_mesh(),
        compiler_params=pltpu.CompilerParams(use_tc_tiling_on_sc=False),
        scratch_types=[pltpu.VMEM((CH,), jnp.int32),
                       pltpu.VMEM((CH, h), jnp.float32),
                       pltpu.SemaphoreType.DMA],
    )
    def k(z_ref, e0_ref, e1_ref, x0_ref, x1_ref, idx_v, rows_v, sem):
        c = lax.axis_index("c")
        s = lax.axis_index("s")
        base = s * (chunks * CH)

        def body(e_ref, x_ref):
            @pl.loop(0, chunks)
            def _(j):
                off = base + j * CH
                pltpu.sync_copy(z_ref.at[pl.ds(off, CH)], idx_v)
                pltpu.async_copy(e_ref.at[idx_v], rows_v, sem).wait()
                pltpu.sync_copy(rows_v, x_ref.at[pl.ds(off, CH)])

        @pl.when(c == 0)
        def _():
            body(e0_ref, x0_ref)

        @pl.when(c == 1)
        def _():
            body(e1_ref, x1_ref)

    return k(z_pad, emb0, emb1)


def _sc_agg(x0, x1, row2d, col2d, n, sp_rows):
    h = x0.shape[1]
    rps = row2d.shape[0] // NS
    nb = rps // BLK
    nout = n // NS

    @functools.partial(
        pl.kernel,
        out_type=[jax.ShapeDtypeStruct((n, h), jnp.float32),
                  jax.ShapeDtypeStruct((n, h), jnp.float32)],
        mesh=_mesh(),
        compiler_params=pltpu.CompilerParams(use_tc_tiling_on_sc=False),
        scratch_types=[pltpu.VMEM((BLK, CH), jnp.int32),
                       pltpu.VMEM((BLK, CH), jnp.int32),
                       pltpu.VMEM((BLK, CH, h), jnp.float32),
                       pltpu.VMEM_SHARED((sp_rows, h), jnp.float32),
                       pltpu.SemaphoreType.DMA,
                       pltpu.SemaphoreType.DMA],
    )
    def k(x0_ref, x1_ref, row_ref, col_ref, o0_ref, o1_ref,
          rowbuf, colbuf, gbuf, acc, sem_g, sem_s):
        c = lax.axis_index("c")
        s = lax.axis_index("s")

        def body(x_ref, o_ref):
            pltpu.sync_copy(x_ref.at[pl.ds(s * nout, nout)],
                            acc.at[pl.ds(s * nout, nout)])
            plsc.subcore_barrier()

            @pl.loop(0, nb)
            def _(b):
                r0 = s * rps + b * BLK
                pltpu.sync_copy(row_ref.at[pl.ds(r0, BLK)], rowbuf)
                pltpu.sync_copy(col_ref.at[pl.ds(r0, BLK)], colbuf)
                gds = [pltpu.async_copy(x_ref.at[colbuf.at[j]],
                                        gbuf.at[j], sem_g)
                       for j in range(BLK)]
                for dsc in gds:
                    dsc.wait()
                sds = [pltpu.async_copy(
                    gbuf.at[j], acc.at[rowbuf.at[j]], sem_s, add=True)
                       for j in range(BLK)]
                for dsc in sds:
                    dsc.wait()

            plsc.subcore_barrier()
            pltpu.sync_copy(acc.at[pl.ds(s * nout, nout)],
                            o_ref.at[pl.ds(s * nout, nout)])

        @pl.when(c == 0)
        def _():
            body(x0_ref, o0_ref)

        @pl.when(c == 1)
        def _():
            body(x1_ref, o1_ref)

    return k(x0, x1, row2d, col2d)


def _tc_linear(s0, s1, w0t, w1t, b2d, final):
    n, h = s0.shape
    d = w0t.shape[1]
    rb = 1000
    grid = (n // rb,)

    def compute(s0r, s1r, w0r, w1r, br):
        y = (jnp.dot(s0r[...], w0r[...],
                     preferred_element_type=jnp.float32)
             + jnp.dot(s1r[...], w1r[...],
                       preferred_element_type=jnp.float32)
             + br[...])
        return jnp.maximum(y, 0.0)

    in_specs = (
        [pl.BlockSpec((rb, h), lambda i: (i, 0)) for _ in range(2)]
        + [pl.BlockSpec((h, d), lambda i: (0, 0)) for _ in range(2)]
        + [pl.BlockSpec((1, d), lambda i: (0, 0))]
    )

    if final:
        def body(s0r, s1r, w0r, w1r, br, o):
            o[...] = compute(s0r, s1r, w0r, w1r, br)

        return pl.pallas_call(
            body, grid=grid, in_specs=in_specs,
            out_specs=pl.BlockSpec((rb, d), lambda i: (i, 0)),
            out_shape=jax.ShapeDtypeStruct((n, d), jnp.float32),
        )(s0, s1, w0t, w1t, b2d)

    def body2(s0r, s1r, w0r, w1r, br, o0, o1):
        y = compute(s0r, s1r, w0r, w1r, br)
        o0[...] = y[:, :h]
        o1[...] = y[:, h:]

    return pl.pallas_call(
        body2, grid=grid, in_specs=in_specs,
        out_specs=[pl.BlockSpec((rb, h), lambda i: (i, 0)),
                   pl.BlockSpec((rb, h), lambda i: (i, 0))],
        out_shape=[jax.ShapeDtypeStruct((n, h), jnp.float32),
                   jax.ShapeDtypeStruct((n, h), jnp.float32)],
    )(s0, s1, w0t, w1t, b2d)


def kernel(z, edge_index, emb, W0, b0, W1, b1, W2, b2):
    n = z.shape[0]
    e = edge_index.shape[1]
    d = emb.shape[1]
    h = d // 2

    n_pad = _round_up(n, NS * CH)
    z_pad = jnp.concatenate(
        [z.astype(jnp.int32), jnp.zeros((n_pad - n,), jnp.int32)])
    x0p, x1p = _sc_embed(z_pad, emb[:, :h], emb[:, h:])
    x0, x1 = x0p[:n], x1p[:n]

    eps = _round_up(-(-e // NS), BLK * CH)
    e_pad = eps * NS
    row = edge_index[0].astype(jnp.int32)
    col = edge_index[1].astype(jnp.int32)
    row2d = jnp.concatenate(
        [row, jnp.full((e_pad - e,), n, jnp.int32)]).reshape(-1, CH)
    col2d = jnp.concatenate(
        [col, jnp.zeros((e_pad - e,), jnp.int32)]).reshape(-1, CH)
    sp_rows = _round_up(n + 1, NS)

    for i, (w, b) in enumerate(((W0, b0), (W1, b1), (W2, b2))):
        s0, s1 = _sc_agg(x0, x1, row2d, col2d, n, sp_rows)
        wt = w.T
        if i < 2:
            x0, x1 = _tc_linear(s0, s1, wt[:h], wt[h:],
                                b.reshape(1, d), final=False)
        else:
            return _tc_linear(s0, s1, wt[:h], wt[h:],
                              b.reshape(1, d), final=True)
